# Initial kernel scaffold; baseline (speedup 1.0000x reference)
import jax
import jax.numpy as jnp
from jax.experimental import pallas as pl


def kernel(hidden_states, Wg, W1, b1, W2, b2):
    raise NotImplementedError("write your pallas kernel here")


# speedup vs baseline: 3.6902x; 3.6902x over previous
import functools
import math

import jax
import jax.numpy as jnp
from jax import lax
from jax.experimental import pallas as pl
from jax.experimental.pallas import tpu as pltpu
from jax.experimental.pallas import tpu_sc as plsc

N_TOK = 8192
D_MODEL = 1024
D_FF = 4096
N_EXP = 8
TOPK = 2
CAP = 2048
DUMMY = N_EXP * CAP
XG_ROWS = N_EXP * CAP + CAP

TOK_BLK = 256
N_TOK_BLKS = N_TOK // TOK_BLK
FF_BLK = 1024
N_FF_BLKS = D_FF // FF_BLK
ROW_BLK = 256

SC_CHUNK = 64
_RSQRT2 = 1.0 / math.sqrt(2.0)


def _router_body(x_ref, wg_ref, s1_ref, s2_ref, w1_ref, w2_ref, carry_ref):
    i = pl.program_id(0)

    @pl.when(i == 0)
    def _():
        carry_ref[...] = jnp.zeros((1, N_EXP), jnp.float32)

    x = x_ref[...]
    wg = wg_ref[...]
    logits = lax.dot_general(
        x, wg, (((1,), (1,)), ((), ())), preferred_element_type=jnp.float32
    )
    z = logits - jnp.max(logits, axis=1, keepdims=True)
    p = jnp.exp(z)
    gates = p / jnp.sum(p, axis=1, keepdims=True)

    e_iota = lax.broadcasted_iota(jnp.int32, (TOK_BLK, N_EXP), 1)
    v1 = jnp.max(gates, axis=1, keepdims=True)
    i1 = jnp.min(jnp.where(gates == v1, e_iota, N_EXP), axis=1, keepdims=True)
    oh1 = e_iota == i1
    g2 = jnp.where(oh1, -1.0, gates)
    v2 = jnp.max(g2, axis=1, keepdims=True)
    i2 = jnp.min(jnp.where(g2 == v2, e_iota, N_EXP), axis=1, keepdims=True)
    oh2 = e_iota == i2

    mf = (oh1 | oh2).astype(jnp.float32)
    row = lax.broadcasted_iota(jnp.int32, (TOK_BLK, TOK_BLK), 0)
    col = lax.broadcasted_iota(jnp.int32, (TOK_BLK, TOK_BLK), 1)
    trif = (row >= col).astype(jnp.float32)
    incl = lax.dot_general(
        trif, mf, (((1,), (0,)), ((), ())), preferred_element_type=jnp.float32
    )
    pos = carry_ref[...] + incl - 1.0
    keepf = jnp.where(pos < float(CAP), mf, 0.0)
    gsf = e_iota.astype(jnp.float32) * float(CAP) + pos
    gs = jnp.where(keepf > 0, gsf, float(DUMMY))

    slot1 = jnp.sum(jnp.where(oh1, gs, 0.0), axis=1, keepdims=True)
    slot2 = jnp.sum(jnp.where(oh2, gs, 0.0), axis=1, keepdims=True)
    k1 = jnp.sum(jnp.where(oh1, keepf, 0.0), axis=1, keepdims=True)
    k2 = jnp.sum(jnp.where(oh2, keepf, 0.0), axis=1, keepdims=True)

    s1_ref[...] = slot1.astype(jnp.int32)
    s2_ref[...] = slot2.astype(jnp.int32)
    w1_ref[...] = v1 * k1
    w2_ref[...] = v2 * k2
    carry_ref[...] = carry_ref[...] + jnp.sum(mf, axis=0, keepdims=True)


def _router(flat, wg):
    return pl.pallas_call(
        _router_body,
        grid=(N_TOK_BLKS,),
        in_specs=[
            pl.BlockSpec((TOK_BLK, D_MODEL), lambda i: (i, 0)),
            pl.BlockSpec((N_EXP, D_MODEL), lambda i: (0, 0)),
        ],
        out_specs=[
            pl.BlockSpec((TOK_BLK, 1), lambda i: (i, 0)),
            pl.BlockSpec((TOK_BLK, 1), lambda i: (i, 0)),
            pl.BlockSpec((TOK_BLK, 1), lambda i: (i, 0)),
            pl.BlockSpec((TOK_BLK, 1), lambda i: (i, 0)),
        ],
        out_shape=[
            jax.ShapeDtypeStruct((N_TOK, 1), jnp.int32),
            jax.ShapeDtypeStruct((N_TOK, 1), jnp.int32),
            jax.ShapeDtypeStruct((N_TOK, 1), jnp.float32),
            jax.ShapeDtypeStruct((N_TOK, 1), jnp.float32),
        ],
        scratch_shapes=[pltpu.VMEM((1, N_EXP), jnp.float32)],
    )(flat, wg)


def _dispatch_body(flat_hbm, s1_hbm, s2_hbm, xg_hbm, idx1, idx2, rows, sem):
    wid = lax.axis_index("c") * 16 + lax.axis_index("s")
    base = wid * (N_TOK // 32)

    @pl.loop(0, (N_TOK // 32) // SC_CHUNK)
    def _(k):
        tok0 = base + k * SC_CHUNK
        pltpu.sync_copy(flat_hbm.at[pl.ds(tok0, SC_CHUNK)], rows)
        pltpu.sync_copy(s1_hbm.at[pl.ds(tok0, SC_CHUNK)], idx1)
        pltpu.sync_copy(s2_hbm.at[pl.ds(tok0, SC_CHUNK)], idx2)
        c1 = pltpu.async_copy(rows, xg_hbm.at[idx1], sem)
        c1.wait()
        c2 = pltpu.async_copy(rows, xg_hbm.at[idx2], sem)
        c2.wait()


def _dispatch(flat, s1, s2):
    mesh = plsc.VectorSubcoreMesh(core_axis_name="c", subcore_axis_name="s")
    k = pl.kernel(
        _dispatch_body,
        out_type=jax.ShapeDtypeStruct((XG_ROWS, D_MODEL), jnp.float32),
        mesh=mesh,
        scratch_types=[
            pltpu.VMEM((SC_CHUNK,), jnp.int32),
            pltpu.VMEM((SC_CHUNK,), jnp.int32),
            pltpu.VMEM((SC_CHUNK, D_MODEL), jnp.float32),
            pltpu.SemaphoreType.DMA,
        ],
    )
    return k(flat, s1, s2)


def _ffn_body(xg_ref, w1_ref, b1_ref, w2_ref, b2_ref, out_ref, xbf):
    f = pl.program_id(1)

    @pl.when(f == 0)
    def _():
        xbf[...] = xg_ref[...].astype(jnp.bfloat16)

    w1b = w1_ref[0].astype(jnp.bfloat16)
    w2b = w2_ref[0].astype(jnp.bfloat16)
    b1v = b1_ref[0]
    b2v = b2_ref[0]
    for c in range(CAP // ROW_BLK):
        xa = xbf[pl.ds(c * ROW_BLK, ROW_BLK), :]
        h = lax.dot_general(
            xa, w1b, (((1,), (0,)), ((), ())), preferred_element_type=jnp.float32
        )
        h = h + b1v
        h = 0.5 * h * (1.0 + lax.erf(h * _RSQRT2))
        y = lax.dot_general(
            h.astype(jnp.bfloat16),
            w2b,
            (((1,), (0,)), ((), ())),
            preferred_element_type=jnp.float32,
        )

        @pl.when(f == 0)
        def _():
            out_ref[pl.ds(c * ROW_BLK, ROW_BLK), :] = y + b2v

        @pl.when(f != 0)
        def _():
            out_ref[pl.ds(c * ROW_BLK, ROW_BLK), :] += y


def _ffn(xg, w1, b1, w2, b2):
    return pl.pallas_call(
        _ffn_body,
        grid=(N_EXP, N_FF_BLKS),
        in_specs=[
            pl.BlockSpec((CAP, D_MODEL), lambda e, f: (e, 0)),
            pl.BlockSpec((1, D_MODEL, FF_BLK), lambda e, f: (e, 0, f)),
            pl.BlockSpec((1, 1, FF_BLK), lambda e, f: (e, 0, f)),
            pl.BlockSpec((1, FF_BLK, D_MODEL), lambda e, f: (e, f, 0)),
            pl.BlockSpec((1, 1, D_MODEL), lambda e, f: (e, 0, 0)),
        ],
        out_specs=pl.BlockSpec((CAP, D_MODEL), lambda e, f: (e, 0)),
        out_shape=jax.ShapeDtypeStruct((XG_ROWS, D_MODEL), jnp.float32),
        scratch_shapes=[pltpu.VMEM((CAP, D_MODEL), jnp.bfloat16)],
    )(xg, w1, b1.reshape(N_EXP, 1, D_FF), w2, b2.reshape(N_EXP, 1, D_MODEL))


def _gather_body(out_hbm, s1_hbm, s2_hbm, g1_hbm, g2_hbm, idx1, idx2, rows, sem):
    wid = lax.axis_index("c") * 16 + lax.axis_index("s")
    base = wid * (N_TOK // 32)

    @pl.loop(0, (N_TOK // 32) // SC_CHUNK)
    def _(k):
        tok0 = base + k * SC_CHUNK
        pltpu.sync_copy(s1_hbm.at[pl.ds(tok0, SC_CHUNK)], idx1)
        pltpu.sync_copy(s2_hbm.at[pl.ds(tok0, SC_CHUNK)], idx2)
        c1 = pltpu.async_copy(out_hbm.at[idx1], rows, sem)
        c1.wait()
        pltpu.sync_copy(rows, g1_hbm.at[pl.ds(tok0, SC_CHUNK)])
        c2 = pltpu.async_copy(out_hbm.at[idx2], rows, sem)
        c2.wait()
        pltpu.sync_copy(rows, g2_hbm.at[pl.ds(tok0, SC_CHUNK)])


def _gather2(out_ffn, s1, s2):
    mesh = plsc.VectorSubcoreMesh(core_axis_name="c", subcore_axis_name="s")
    k = pl.kernel(
        _gather_body,
        out_type=(
            jax.ShapeDtypeStruct((N_TOK, D_MODEL), jnp.float32),
            jax.ShapeDtypeStruct((N_TOK, D_MODEL), jnp.float32),
        ),
        mesh=mesh,
        scratch_types=[
            pltpu.VMEM((SC_CHUNK,), jnp.int32),
            pltpu.VMEM((SC_CHUNK,), jnp.int32),
            pltpu.VMEM((SC_CHUNK, D_MODEL), jnp.float32),
            pltpu.SemaphoreType.DMA,
        ],
    )
    return k(out_ffn, s1, s2)


def _combine_body(g1_ref, g2_ref, w1_ref, w2_ref, o_ref):
    w1v = w1_ref[...]
    w2v = w2_ref[...]
    a = jnp.where(w1v > 0, g1_ref[...], 0.0) * w1v
    b = jnp.where(w2v > 0, g2_ref[...], 0.0) * w2v
    o_ref[...] = a + b


def _combine(g1, g2, w1, w2):
    return pl.pallas_call(
        _combine_body,
        grid=(N_TOK_BLKS,),
        in_specs=[
            pl.BlockSpec((TOK_BLK, D_MODEL), lambda i: (i, 0)),
            pl.BlockSpec((TOK_BLK, D_MODEL), lambda i: (i, 0)),
            pl.BlockSpec((TOK_BLK, 1), lambda i: (i, 0)),
            pl.BlockSpec((TOK_BLK, 1), lambda i: (i, 0)),
        ],
        out_specs=pl.BlockSpec((TOK_BLK, D_MODEL), lambda i: (i, 0)),
        out_shape=jax.ShapeDtypeStruct((N_TOK, D_MODEL), jnp.float32),
    )(g1, g2, w1, w2)


def kernel(hidden_states, Wg, W1, b1, W2, b2):
    bh, th, d = hidden_states.shape
    flat = hidden_states.reshape(bh * th, d)
    s1, s2, w1, w2 = _router(flat, Wg)
    s1f = s1.reshape(N_TOK)
    s2f = s2.reshape(N_TOK)
    xg = _dispatch(flat, s1f, s2f)
    out_ffn = _ffn(xg, W1, b1, W2, b2)
    g1, g2 = _gather2(out_ffn, s1f, s2f)
    final = _combine(g1, g2, w1, w2)
    aux_loss = jnp.asarray(0.0, dtype=jnp.float32)
    return final.reshape(bh, th, d), aux_loss
